# baseline (speedup 1.0000x reference)
import jax
import jax.numpy as jnp
from jax import lax
from jax.experimental import pallas as pl
from jax.experimental.pallas import tpu as pltpu
from jax.experimental.pallas import tpu_sc as plsc

B = 1024
SEQ = 77
N_CTX = 4
D = 512
TOK = SEQ - N_CTX
TOK_PAD = 80
NROW = 88
NC = 2
NS = 16
NW = NC * NS
BPW = B // NW
GRP = BPW // 8
UNROLL = 4
NIT = BPW // UNROLL


def _sc_kernel(gidx_hbm, sidx_hbm, ctx_hbm, table_hbm, out_hbm,
               gidx_v, sidx_v, buf0, buf1, gsem0, gsem1, ssem0, ssem1):
    wid = lax.axis_index("s") * NC + lax.axis_index("c")
    base = wid * BPW

    def start_gather(j, buf, sem):
        idx = gidx_v.at[jnp.right_shift(j, 3), jnp.bitwise_and(j, 7)]
        pltpu.async_copy(table_hbm.at[idx], buf.at[pl.ds(0, TOK_PAD)], sem)

    def wait_gather(j, buf, sem):
        idx = gidx_v.at[jnp.right_shift(j, 3), jnp.bitwise_and(j, 7)]
        pltpu.make_async_copy(table_hbm.at[idx],
                              buf.at[pl.ds(0, TOK_PAD)], sem).wait()

    def start_scatter(j, buf, sem):
        pltpu.async_copy(buf, out_hbm.at[base + j].at[sidx_v.at[0]], sem)

    def wait_scatter(j, buf, sem):
        pltpu.make_async_copy(buf, out_hbm.at[base + j].at[sidx_v.at[0]],
                              sem).wait()

    pltpu.sync_copy(gidx_hbm.at[pl.ds(wid * GRP, GRP)], gidx_v)
    pltpu.sync_copy(sidx_hbm, sidx_v)
    for buf in (buf0, buf1):
        pltpu.sync_copy(ctx_hbm, buf.at[pl.ds(TOK_PAD, N_CTX)])
        pltpu.sync_copy(ctx_hbm, buf.at[pl.ds(TOK_PAD + N_CTX, N_CTX)])

    start_gather(0, buf0, gsem0)

    def body(t, carry):
        j = 2 * t
        wait_gather(j, buf0, gsem0)
        start_scatter(j, buf0, ssem0)

        @pl.when(t > 0)
        def _():
            wait_scatter(j - 1, buf1, ssem1)

        start_gather(j + 1, buf1, gsem1)
        wait_gather(j + 1, buf1, gsem1)
        start_scatter(j + 1, buf1, ssem1)
        wait_scatter(j, buf0, ssem0)

        @pl.when(t < BPW // 2 - 1)
        def _():
            start_gather(j + 2, buf0, gsem0)

        return carry

    lax.fori_loop(0, BPW // 2, body, 0)
    wait_scatter(BPW - 1, buf1, ssem1)


def kernel(text, ctx, token_embedding):
    gidx = jnp.pad(text[:, :TOK], ((0, 0), (0, TOK_PAD - TOK)), mode="edge")
    tok_rows = N_CTX + jnp.arange(TOK, dtype=jnp.int32)
    dup_rows = jnp.full((TOK_PAD - TOK,), SEQ - 1, jnp.int32)
    ctx_rows = jnp.arange(N_CTX, dtype=jnp.int32)
    sidx = jnp.concatenate([tok_rows, dup_rows, ctx_rows, ctx_rows])

    mesh = plsc.VectorSubcoreMesh(core_axis_name="c", subcore_axis_name="s",
                                  num_cores=NC, num_subcores=NS)
    run = pl.kernel(
        _sc_kernel,
        out_type=jax.ShapeDtypeStruct((B, SEQ, D), jnp.float32),
        mesh=mesh,
        scratch_types=(
            [pltpu.VMEM((GRP, 8, TOK_PAD), jnp.int32)]
            + [pltpu.VMEM((1, NROW), jnp.int32)]
            + [pltpu.VMEM((NROW, D), jnp.float32)] * 2
            + [pltpu.SemaphoreType.DMA] * 4
        ),
    )
    return run(gidx.reshape(B // 8, 8, TOK_PAD), sidx.reshape(1, NROW),
               ctx, token_embedding)


# speedup vs baseline: 1.0594x; 1.0594x over previous
import jax
import jax.numpy as jnp
from jax import lax
from jax.experimental import pallas as pl
from jax.experimental.pallas import tpu as pltpu
from jax.experimental.pallas import tpu_sc as plsc

B = 1024
SEQ = 77
N_CTX = 4
D = 512
TOK = SEQ - N_CTX
TOK_PAD = 80
NROW = 88
NC = 2
NS = 16
NW = NC * NS
BPW = B // NW
GRP = BPW // 8
UNROLL = 4
NIT = BPW // UNROLL


def _sc_kernel(gidx_hbm, sidx_hbm, ctx_hbm, table_hbm, out_hbm,
               gidx_v, sidx_v, ctx_v, buf0, buf1,
               gsem0, gsem1, ssem0, ssem1, csem):
    wid = lax.axis_index("s") * NC + lax.axis_index("c")
    base = wid * BPW

    def start_gather(j, buf, sem):
        idx = gidx_v.at[jnp.right_shift(j, 3), jnp.bitwise_and(j, 7)]
        pltpu.async_copy(table_hbm.at[idx], buf, sem)

    def wait_gather(j, buf, sem):
        idx = gidx_v.at[jnp.right_shift(j, 3), jnp.bitwise_and(j, 7)]
        pltpu.make_async_copy(table_hbm.at[idx], buf, sem).wait()

    def start_scatter(j, buf, sem):
        pltpu.async_copy(buf, out_hbm.at[base + j].at[sidx_v.at[0]], sem)

    def wait_scatter(j, buf, sem):
        pltpu.make_async_copy(buf, out_hbm.at[base + j].at[sidx_v.at[0]],
                              sem).wait()

    def start_ctx(j):
        pltpu.async_copy(ctx_v, out_hbm.at[base + j, pl.ds(0, N_CTX)], csem)

    def wait_ctx(j):
        pltpu.make_async_copy(ctx_v, out_hbm.at[base + j, pl.ds(0, N_CTX)],
                              csem).wait()

    pltpu.sync_copy(gidx_hbm.at[pl.ds(wid * GRP, GRP)], gidx_v)
    pltpu.sync_copy(sidx_hbm, sidx_v)
    pltpu.sync_copy(ctx_hbm, ctx_v)

    start_gather(0, buf0, gsem0)
    start_gather(1, buf1, gsem1)
    for j in range(UNROLL):
        start_ctx(j)

    def body(u, carry):
        j = u * UNROLL
        last = u >= NIT - 1
        wait_gather(j, buf0, gsem0)
        start_scatter(j, buf0, ssem0)
        wait_gather(j + 1, buf1, gsem1)
        start_scatter(j + 1, buf1, ssem1)
        wait_scatter(j, buf0, ssem0)
        start_gather(j + 2, buf0, gsem0)
        wait_scatter(j + 1, buf1, ssem1)
        start_gather(j + 3, buf1, gsem1)

        wait_gather(j + 2, buf0, gsem0)
        start_scatter(j + 2, buf0, ssem0)
        wait_gather(j + 3, buf1, gsem1)
        start_scatter(j + 3, buf1, ssem1)
        wait_scatter(j + 2, buf0, ssem0)

        @pl.when(jnp.logical_not(last))
        def _():
            start_gather(j + 4, buf0, gsem0)

        wait_scatter(j + 3, buf1, ssem1)

        @pl.when(jnp.logical_not(last))
        def _():
            start_gather(j + 5, buf1, gsem1)
            start_ctx(j + 4)
            start_ctx(j + 5)
            start_ctx(j + 6)
            start_ctx(j + 7)

        wait_ctx(j)
        wait_ctx(j + 1)
        wait_ctx(j + 2)
        wait_ctx(j + 3)

        return carry

    lax.fori_loop(0, NIT, body, 0)


def kernel(text, ctx, token_embedding):
    gidx = jnp.pad(text[:, :TOK], ((0, 0), (0, TOK_PAD - TOK)), mode="edge")
    tok_rows = N_CTX + jnp.arange(TOK, dtype=jnp.int32)
    dup_rows = jnp.full((TOK_PAD - TOK,), SEQ - 1, jnp.int32)
    sidx = jnp.concatenate([tok_rows, dup_rows])

    mesh = plsc.VectorSubcoreMesh(core_axis_name="c", subcore_axis_name="s",
                                  num_cores=NC, num_subcores=NS)
    run = pl.kernel(
        _sc_kernel,
        out_type=jax.ShapeDtypeStruct((B, SEQ, D), jnp.float32),
        mesh=mesh,
        scratch_types=(
            [pltpu.VMEM((GRP, 8, TOK_PAD), jnp.int32)]
            + [pltpu.VMEM((1, TOK_PAD), jnp.int32)]
            + [pltpu.VMEM((N_CTX, D), jnp.float32)]
            + [pltpu.VMEM((TOK_PAD, D), jnp.float32)] * 2
            + [pltpu.SemaphoreType.DMA] * 5
        ),
    )
    return run(gidx.reshape(B // 8, 8, TOK_PAD), sidx.reshape(1, TOK_PAD),
               ctx, token_embedding)
